# baseline (speedup 1.0000x reference)
import jax
import jax.numpy as jnp
from jax import lax
from jax.experimental import pallas as pl
from jax.experimental.pallas import tpu as pltpu
from jax.experimental.pallas import tpu_sc as plsc

L_Q = 1024
L_K = 1024
D_MODEL = 64
MAX_K = 16
N_PE = 2 * MAX_K + 1
N_TILES = 32
D_PER_TILE = D_MODEL // N_TILES
ET_COLS = 2176
BQ = 8


def _sc_gather_body(pet_hbm, et_hbm, pet_v, row_v, sem):
    c = lax.axis_index("c")
    s = lax.axis_index("s")
    wid = s * 2 + c

    pltpu.sync_copy(pet_hbm, pet_v)

    for dd in range(D_PER_TILE):
        d = wid * D_PER_TILE + dd
        row = d * N_PE

        def fill(cc, carry, row=row, dd=dd):
            g = lax.iota(jnp.int32, 16) + cc * 16
            sidx = jnp.clip(g - (L_K - 1), -MAX_K, MAX_K) + MAX_K
            row_v[dd, pl.ds(cc * 16, 16)] = plsc.load_gather(pet_v, [row + sidx])
            return carry

        lax.fori_loop(0, ET_COLS // 16, fill, 0)

    writes = [
        pltpu.async_copy(row_v.at[dd], et_hbm.at[wid * D_PER_TILE + dd], sem)
        for dd in range(D_PER_TILE)
    ]
    for w in writes:
        w.wait()


def _tc_window_body(e_ref, out_ref):
    i = pl.program_id(0)
    for j in range(BQ):
        q = i * BQ + j
        off = (L_K - 1) - q
        base = pl.multiple_of((off // 128) * 128, 128)
        m = off % 128
        w = e_ref[:, pl.ds(base, L_K + 128)]
        rolled = pltpu.roll(w, (L_K + 128 - m) % (L_K + 128), 1)
        out_ref[j] = rolled[:, :L_K]


def kernel(length_q, length_k, pe):
    del length_q, length_k

    mesh = plsc.VectorSubcoreMesh(core_axis_name="c", subcore_axis_name="s")
    gather = pl.kernel(
        _sc_gather_body,
        out_type=jax.ShapeDtypeStruct((D_MODEL, ET_COLS), jnp.float32),
        mesh=mesh,
        scratch_types=[
            pltpu.VMEM((D_MODEL * N_PE,), jnp.float32),
            pltpu.VMEM((D_PER_TILE, ET_COLS), jnp.float32),
            pltpu.SemaphoreType.DMA,
        ],
        compiler_params=pltpu.CompilerParams(
            use_tc_tiling_on_sc=False, needs_layout_passes=False
        ),
    )
    et = gather(pe.T.reshape(-1))

    out_qdk = pl.pallas_call(
        _tc_window_body,
        grid=(L_Q // BQ,),
        in_specs=[pl.BlockSpec((D_MODEL, ET_COLS), lambda i: (0, 0))],
        out_specs=pl.BlockSpec((BQ, D_MODEL, L_K), lambda i: (i, 0, 0)),
        out_shape=jax.ShapeDtypeStruct((L_Q, D_MODEL, L_K), jnp.float32),
    )(et)
    return jnp.transpose(out_qdk, (0, 2, 1))


# speedup vs baseline: 1.1770x; 1.1770x over previous
import jax
import jax.numpy as jnp
from jax import lax
from jax.experimental import pallas as pl
from jax.experimental.pallas import tpu as pltpu
from jax.experimental.pallas import tpu_sc as plsc

L_Q = 1024
L_K = 1024
D_MODEL = 64
MAX_K = 16
N_PE = 2 * MAX_K + 1
N_TILES = 32
D_PER_TILE = D_MODEL // N_TILES
ET_COLS = 2176
BQ = 16


def _sc_gather_body(pet_hbm, et_hbm, pet_v, row_v, sem):
    c = lax.axis_index("c")
    s = lax.axis_index("s")
    wid = s * 2 + c

    pltpu.sync_copy(pet_hbm, pet_v)

    for dd in range(D_PER_TILE):
        d = wid * D_PER_TILE + dd
        row = d * N_PE

        def fill(cc, carry, row=row, dd=dd):
            g = lax.iota(jnp.int32, 16) + cc * 16
            sidx = jnp.clip(g - (L_K - 1), -MAX_K, MAX_K) + MAX_K
            row_v[dd, pl.ds(cc * 16, 16)] = plsc.load_gather(pet_v, [row + sidx])
            return carry

        lax.fori_loop(0, ET_COLS // 16, fill, 0)

    writes = [
        pltpu.async_copy(row_v.at[dd], et_hbm.at[wid * D_PER_TILE + dd], sem)
        for dd in range(D_PER_TILE)
    ]
    for w in writes:
        w.wait()


def _tc_window_body(e_ref, out_ref):
    i = pl.program_id(0)
    for j in range(BQ):
        q = i * BQ + j
        off = (L_K - 1) - q
        base = pl.multiple_of((off // 128) * 128, 128)
        m = off % 128
        w = e_ref[:, pl.ds(base, L_K + 128)]
        rolled = pltpu.roll(w, (L_K + 128 - m) % (L_K + 128), 1)
        out_ref[j] = rolled[:, :L_K]


def kernel(length_q, length_k, pe):
    del length_q, length_k

    mesh = plsc.VectorSubcoreMesh(core_axis_name="c", subcore_axis_name="s")
    gather = pl.kernel(
        _sc_gather_body,
        out_type=jax.ShapeDtypeStruct((D_MODEL, ET_COLS), jnp.float32),
        mesh=mesh,
        scratch_types=[
            pltpu.VMEM((D_MODEL * N_PE,), jnp.float32),
            pltpu.VMEM((D_PER_TILE, ET_COLS), jnp.float32),
            pltpu.SemaphoreType.DMA,
        ],
        compiler_params=pltpu.CompilerParams(
            use_tc_tiling_on_sc=False, needs_layout_passes=False
        ),
    )
    et = gather(pe.T.reshape(-1))

    out_qdk = pl.pallas_call(
        _tc_window_body,
        grid=(L_Q // BQ,),
        in_specs=[pl.BlockSpec((D_MODEL, ET_COLS), lambda i: (0, 0))],
        out_specs=pl.BlockSpec((BQ, D_MODEL, L_K), lambda i: (i, 0, 0)),
        out_shape=jax.ShapeDtypeStruct((L_Q, D_MODEL, L_K), jnp.float32),
    )(et)
    return jnp.transpose(out_qdk, (0, 2, 1))


# speedup vs baseline: 1.2826x; 1.0897x over previous
import jax
import jax.numpy as jnp
from jax import lax
from jax.experimental import pallas as pl
from jax.experimental.pallas import tpu as pltpu
from jax.experimental.pallas import tpu_sc as plsc

L_Q = 1024
L_K = 1024
D_MODEL = 64
MAX_K = 16
N_PE = 2 * MAX_K + 1
N_TILES = 32
D_PER_TILE = D_MODEL // N_TILES
ET_COLS = 2176
BQ = 32


def _sc_gather_body(pet_hbm, et_hbm, pet_v, row_v, sem):
    c = lax.axis_index("c")
    s = lax.axis_index("s")
    wid = s * 2 + c

    pltpu.sync_copy(pet_hbm, pet_v)

    for dd in range(D_PER_TILE):
        d = wid * D_PER_TILE + dd
        row = d * N_PE

        def fill(cc, carry, row=row, dd=dd):
            g = lax.iota(jnp.int32, 16) + cc * 16
            sidx = jnp.clip(g - (L_K - 1), -MAX_K, MAX_K) + MAX_K
            row_v[dd, pl.ds(cc * 16, 16)] = plsc.load_gather(pet_v, [row + sidx])
            return carry

        lax.fori_loop(0, ET_COLS // 16, fill, 0)

    writes = [
        pltpu.async_copy(row_v.at[dd], et_hbm.at[wid * D_PER_TILE + dd], sem)
        for dd in range(D_PER_TILE)
    ]
    for w in writes:
        w.wait()


def _tc_window_body(e_ref, out_ref):
    i = pl.program_id(0)
    for j in range(BQ):
        q = i * BQ + j
        off = (L_K - 1) - q
        base = pl.multiple_of((off // 128) * 128, 128)
        m = off % 128
        w = e_ref[:, pl.ds(base, L_K + 128)]
        rolled = pltpu.roll(w, (L_K + 128 - m) % (L_K + 128), 1)
        out_ref[j] = rolled[:, :L_K]


def kernel(length_q, length_k, pe):
    del length_q, length_k

    mesh = plsc.VectorSubcoreMesh(core_axis_name="c", subcore_axis_name="s")
    gather = pl.kernel(
        _sc_gather_body,
        out_type=jax.ShapeDtypeStruct((D_MODEL, ET_COLS), jnp.float32),
        mesh=mesh,
        scratch_types=[
            pltpu.VMEM((D_MODEL * N_PE,), jnp.float32),
            pltpu.VMEM((D_PER_TILE, ET_COLS), jnp.float32),
            pltpu.SemaphoreType.DMA,
        ],
        compiler_params=pltpu.CompilerParams(
            use_tc_tiling_on_sc=False, needs_layout_passes=False
        ),
    )
    et = gather(pe.T.reshape(-1))

    out_qdk = pl.pallas_call(
        _tc_window_body,
        grid=(L_Q // BQ,),
        in_specs=[pl.BlockSpec((D_MODEL, ET_COLS), lambda i: (0, 0))],
        out_specs=pl.BlockSpec((BQ, D_MODEL, L_K), lambda i: (i, 0, 0)),
        out_shape=jax.ShapeDtypeStruct((L_Q, D_MODEL, L_K), jnp.float32),
    )(et)
    return jnp.transpose(out_qdk, (0, 2, 1))


# speedup vs baseline: 1.3079x; 1.0197x over previous
import jax
import jax.numpy as jnp
from jax import lax
from jax.experimental import pallas as pl
from jax.experimental.pallas import tpu as pltpu
from jax.experimental.pallas import tpu_sc as plsc

L_Q = 1024
L_K = 1024
D_MODEL = 64
MAX_K = 16
N_PE = 2 * MAX_K + 1
N_TILES = 32
D_PER_TILE = D_MODEL // N_TILES
ET_COLS = 2176
BQ = 64


def _sc_gather_body(pet_hbm, et_hbm, pet_v, row_v, sem):
    c = lax.axis_index("c")
    s = lax.axis_index("s")
    wid = s * 2 + c

    pltpu.sync_copy(pet_hbm, pet_v)

    for dd in range(D_PER_TILE):
        d = wid * D_PER_TILE + dd
        row = d * N_PE

        def fill(cc, carry, row=row, dd=dd):
            g = lax.iota(jnp.int32, 16) + cc * 16
            sidx = jnp.clip(g - (L_K - 1), -MAX_K, MAX_K) + MAX_K
            row_v[dd, pl.ds(cc * 16, 16)] = plsc.load_gather(pet_v, [row + sidx])
            return carry

        lax.fori_loop(0, ET_COLS // 16, fill, 0)

    writes = [
        pltpu.async_copy(row_v.at[dd], et_hbm.at[wid * D_PER_TILE + dd], sem)
        for dd in range(D_PER_TILE)
    ]
    for w in writes:
        w.wait()


def _tc_window_body(e_ref, out_ref):
    i = pl.program_id(0)
    for j in range(BQ):
        q = i * BQ + j
        off = (L_K - 1) - q
        base = pl.multiple_of((off // 128) * 128, 128)
        m = off % 128
        w = e_ref[:, pl.ds(base, L_K + 128)]
        rolled = pltpu.roll(w, (L_K + 128 - m) % (L_K + 128), 1)
        out_ref[j] = rolled[:, :L_K]


def kernel(length_q, length_k, pe):
    del length_q, length_k

    mesh = plsc.VectorSubcoreMesh(core_axis_name="c", subcore_axis_name="s")
    gather = pl.kernel(
        _sc_gather_body,
        out_type=jax.ShapeDtypeStruct((D_MODEL, ET_COLS), jnp.float32),
        mesh=mesh,
        scratch_types=[
            pltpu.VMEM((D_MODEL * N_PE,), jnp.float32),
            pltpu.VMEM((D_PER_TILE, ET_COLS), jnp.float32),
            pltpu.SemaphoreType.DMA,
        ],
        compiler_params=pltpu.CompilerParams(
            use_tc_tiling_on_sc=False, needs_layout_passes=False
        ),
    )
    et = gather(pe.T.reshape(-1))

    out_qdk = pl.pallas_call(
        _tc_window_body,
        grid=(L_Q // BQ,),
        in_specs=[pl.BlockSpec((D_MODEL, ET_COLS), lambda i: (0, 0))],
        out_specs=pl.BlockSpec((BQ, D_MODEL, L_K), lambda i: (i, 0, 0)),
        out_shape=jax.ShapeDtypeStruct((L_Q, D_MODEL, L_K), jnp.float32),
    )(et)
    return jnp.transpose(out_qdk, (0, 2, 1))
